# Initial kernel scaffold; baseline (speedup 1.0000x reference)
import jax
import jax.numpy as jnp
from jax.experimental import pallas as pl


def kernel(x, year_w, month_w, day_w, weekday_w, hour_w, min_w):
    raise NotImplementedError("write your pallas kernel here")


# speedup vs baseline: 5.5420x; 5.5420x over previous
import functools

import jax
import jax.numpy as jnp
from jax import lax
from jax.experimental import pallas as pl
from jax.experimental.pallas import tpu as pltpu
from jax.experimental.pallas import tpu_sc as plsc

_NC = 2
_NS = 16
_L = 16
_NW = _NC * _NS

_YEARS = 14
_YEAR0 = 2010
_EMB = 128

_MD = 12 * 31
_WH = 7 * 24
_YM = _YEARS * 6
_ROWS = _MD + _WH + _YM

_CHUNK = 256


def _build_pairs(tbl_v, a_v, b_v, dst_off, nb, count):
    def body(r, _):
        i = r // nb
        j = r - i * nb
        for k in range(_EMB // _L):
            va = a_v[pl.ds(i * _EMB + k * _L, _L)]
            vb = b_v[pl.ds(j * _EMB + k * _L, _L)]
            tbl_v[pl.ds((dst_off + r) * _EMB + k * _L, _L)] = va + vb
        return 0

    lax.fori_loop(0, count, body, 0)


def _sc_lookup(x_flat, yw, mw, dw, wdw, hw, nw):
    npos = x_flat.shape[0] // 7
    per_w = npos // _NW
    nchunk = per_w // _CHUNK
    mesh = plsc.VectorSubcoreMesh(core_axis_name="c", subcore_axis_name="s")

    @functools.partial(
        pl.kernel,
        mesh=mesh,
        compiler_params=pltpu.CompilerParams(needs_layout_passes=False),
        out_type=jax.ShapeDtypeStruct((npos * _EMB,), jnp.float32),
        scratch_types=[
            pltpu.VMEM((_ROWS * _EMB,), jnp.float32),
            pltpu.VMEM((_YEARS * _EMB,), jnp.float32),
            pltpu.VMEM((12 * _EMB,), jnp.float32),
            pltpu.VMEM((31 * _EMB,), jnp.float32),
            pltpu.VMEM((7 * _EMB,), jnp.float32),
            pltpu.VMEM((24 * _EMB,), jnp.float32),
            pltpu.VMEM((6 * _EMB,), jnp.float32),
            pltpu.VMEM((_CHUNK * 7,), jnp.int32),
            pltpu.VMEM((_CHUNK * _EMB,), jnp.float32),
        ],
    )
    def body(x_hbm, yw_hbm, mw_hbm, dw_hbm, wdw_hbm, hw_hbm, nw_hbm, out_hbm,
             tbl_v, yv, mv, dv, wv, hv, nv, x_v, o_v):
        wid = lax.axis_index("s") * _NC + lax.axis_index("c")
        base = wid * per_w

        pltpu.sync_copy(yw_hbm, yv)
        pltpu.sync_copy(mw_hbm, mv)
        pltpu.sync_copy(dw_hbm, dv)
        pltpu.sync_copy(wdw_hbm, wv)
        pltpu.sync_copy(hw_hbm, hv)
        pltpu.sync_copy(nw_hbm, nv)

        _build_pairs(tbl_v, mv, dv, 0, 31, _MD)
        _build_pairs(tbl_v, wv, hv, _MD, 24, _WH)
        _build_pairs(tbl_v, yv, nv, _MD + _WH, 6, _YM)

        lane = lax.iota(jnp.int32, _L)

        def chunk_body(t, _):
            pos0 = base + t * _CHUNK
            pltpu.sync_copy(x_hbm.at[pl.ds(pos0 * 7, _CHUNK * 7)], x_v)
            for g in range(_CHUNK // _L):
                p7 = (g * _L) * 7 + lane * 7
                year = plsc.load_gather(x_v, [p7])
                month = plsc.load_gather(x_v, [p7 + 1])
                day = plsc.load_gather(x_v, [p7 + 2])
                wday = plsc.load_gather(x_v, [p7 + 3])
                hour = plsc.load_gather(x_v, [p7 + 4])
                minute = plsc.load_gather(x_v, [p7 + 5])
                in_range = (year >= _YEAR0) & (year <= _YEAR0 + _YEARS - 1)
                yi = jnp.where(in_range, year - _YEAR0, year)
                yi = jnp.minimum(jnp.maximum(yi, 0), _YEARS - 1)
                mi = jnp.minimum(jnp.maximum(month - 1, 0), 11)
                di = jnp.minimum(jnp.maximum(day - 1, 0), 30)
                wi = jnp.minimum(jnp.maximum(wday, 0), 6)
                hi = jnp.minimum(jnp.maximum(hour, 0), 23)
                ni = jnp.minimum(jnp.maximum(lax.div(minute, 10), 0), 5)
                md = (mi * 31 + di) * _EMB
                wh = (wi * 24 + hi + _MD) * _EMB
                ym = (yi * 6 + ni + _MD + _WH) * _EMB
                ob = (g * _L + lane) * _EMB

                def col_body(c, _):
                    v = (plsc.load_gather(tbl_v, [md + c])
                         + plsc.load_gather(tbl_v, [wh + c])
                         + plsc.load_gather(tbl_v, [ym + c]))
                    plsc.store_scatter(o_v, [ob + c], v)
                    return 0

                lax.fori_loop(0, _EMB, col_body, 0)
            pltpu.sync_copy(o_v, out_hbm.at[pl.ds(pos0 * _EMB, _CHUNK * _EMB)])
            return 0

        lax.fori_loop(0, nchunk, chunk_body, 0)

    return body(x_flat, yw, mw, dw, wdw, hw, nw)


def kernel(x, year_w, month_w, day_w, weekday_w, hour_w, min_w):
    b, l, _ = x.shape
    out_flat = _sc_lookup(
        x.reshape(-1),
        year_w.reshape(-1), month_w.reshape(-1), day_w.reshape(-1),
        weekday_w.reshape(-1), hour_w.reshape(-1), min_w.reshape(-1),
    )
    return out_flat.reshape(b, l, _EMB)


# speedup vs baseline: 7.6805x; 1.3859x over previous
import functools

import jax
import jax.numpy as jnp
from jax import lax
from jax.experimental import pallas as pl
from jax.experimental.pallas import tpu as pltpu
from jax.experimental.pallas import tpu_sc as plsc

_NC = 2
_NS = 16
_L = 16
_NW = _NC * _NS

_YEARS = 14
_YEAR0 = 2010
_EMB = 128

_MD = 12 * 31
_WH = 7 * 24
_YM = _YEARS * 6
_ROWS = _MD + _WH + _YM

_CHUNK = 256


def _build_pairs(tbl_v, a_v, b_v, dst_off, nb, count):
    def body(r, _):
        i = r // nb
        j = r - i * nb
        for k in range(_EMB // _L):
            va = a_v[pl.ds(i * _EMB + k * _L, _L)]
            vb = b_v[pl.ds(j * _EMB + k * _L, _L)]
            tbl_v[pl.ds((dst_off + r) * _EMB + k * _L, _L)] = va + vb
        return 0

    lax.fori_loop(0, count, body, 0)


def _sc_lookup(x_flat, yw, mw, dw, wdw, hw, nw):
    npos = x_flat.shape[0] // 7
    per_w = npos // _NW
    nchunk = per_w // _CHUNK
    mesh = plsc.VectorSubcoreMesh(core_axis_name="c", subcore_axis_name="s")

    @functools.partial(
        pl.kernel,
        mesh=mesh,
        compiler_params=pltpu.CompilerParams(needs_layout_passes=False),
        out_type=jax.ShapeDtypeStruct((npos * _EMB,), jnp.float32),
        scratch_types=[
            pltpu.VMEM((_ROWS * _EMB,), jnp.float32),
            pltpu.VMEM((_YEARS * _EMB,), jnp.float32),
            pltpu.VMEM((12 * _EMB,), jnp.float32),
            pltpu.VMEM((31 * _EMB,), jnp.float32),
            pltpu.VMEM((7 * _EMB,), jnp.float32),
            pltpu.VMEM((24 * _EMB,), jnp.float32),
            pltpu.VMEM((6 * _EMB,), jnp.float32),
            pltpu.VMEM((_CHUNK * 7,), jnp.int32),
            pltpu.VMEM((_CHUNK * _EMB,), jnp.float32),
        ],
    )
    def body(x_hbm, yw_hbm, mw_hbm, dw_hbm, wdw_hbm, hw_hbm, nw_hbm, out_hbm,
             tbl_v, yv, mv, dv, wv, hv, nv, x_v, o_v):
        wid = lax.axis_index("s") * _NC + lax.axis_index("c")
        base = wid * per_w

        pltpu.sync_copy(yw_hbm, yv)
        pltpu.sync_copy(mw_hbm, mv)
        pltpu.sync_copy(dw_hbm, dv)
        pltpu.sync_copy(wdw_hbm, wv)
        pltpu.sync_copy(hw_hbm, hv)
        pltpu.sync_copy(nw_hbm, nv)

        _build_pairs(tbl_v, mv, dv, 0, 31, _MD)
        _build_pairs(tbl_v, wv, hv, _MD, 24, _WH)
        _build_pairs(tbl_v, yv, nv, _MD + _WH, 6, _YM)

        lane = lax.iota(jnp.int32, _L)

        def chunk_body(t, _):
            pos0 = base + t * _CHUNK
            pltpu.sync_copy(x_hbm.at[pl.ds(pos0 * 7, _CHUNK * 7)], x_v)
            for g in range(_CHUNK // _L):
                p7 = (g * _L) * 7 + lane * 7
                year = plsc.load_gather(x_v, [p7])
                month = plsc.load_gather(x_v, [p7 + 1])
                day = plsc.load_gather(x_v, [p7 + 2])
                wday = plsc.load_gather(x_v, [p7 + 3])
                hour = plsc.load_gather(x_v, [p7 + 4])
                minute = plsc.load_gather(x_v, [p7 + 5])
                in_range = (year >= _YEAR0) & (year <= _YEAR0 + _YEARS - 1)
                yi = jnp.where(in_range, year - _YEAR0, year)
                yi = jnp.minimum(jnp.maximum(yi, 0), _YEARS - 1)
                mi = jnp.minimum(jnp.maximum(month - 1, 0), 11)
                di = jnp.minimum(jnp.maximum(day - 1, 0), 30)
                wi = jnp.minimum(jnp.maximum(wday, 0), 6)
                hi = jnp.minimum(jnp.maximum(hour, 0), 23)
                ni = jnp.minimum(jnp.maximum(lax.div(minute, 10), 0), 5)
                md = (mi * 31 + di) * _EMB
                wh = (wi * 24 + hi + _MD) * _EMB
                ym = (yi * 6 + ni + _MD + _WH) * _EMB
                ob = (g * _L + lane) * _EMB

                @plsc.parallel_loop(0, _EMB, 1, unroll=8)
                def col_body(c):
                    v = (plsc.load_gather(tbl_v, [md + c])
                         + plsc.load_gather(tbl_v, [wh + c])
                         + plsc.load_gather(tbl_v, [ym + c]))
                    plsc.store_scatter(o_v, [ob + c], v)
            pltpu.sync_copy(o_v, out_hbm.at[pl.ds(pos0 * _EMB, _CHUNK * _EMB)])
            return 0

        lax.fori_loop(0, nchunk, chunk_body, 0)

    return body(x_flat, yw, mw, dw, wdw, hw, nw)


def kernel(x, year_w, month_w, day_w, weekday_w, hour_w, min_w):
    b, l, _ = x.shape
    out_flat = _sc_lookup(
        x.reshape(-1),
        year_w.reshape(-1), month_w.reshape(-1), day_w.reshape(-1),
        weekday_w.reshape(-1), hour_w.reshape(-1), min_w.reshape(-1),
    )
    return out_flat.reshape(b, l, _EMB)


# speedup vs baseline: 26.9804x; 3.5128x over previous
import functools

import jax
import jax.numpy as jnp
from jax import lax
from jax.experimental import pallas as pl
from jax.experimental.pallas import tpu as pltpu
from jax.experimental.pallas import tpu_sc as plsc

_NC = 2
_NS = 16
_L = 16
_NW = _NC * _NS

_YEARS = 14
_YEAR0 = 2010
_EMB = 128

_MD = 12 * 31
_WH = 7 * 24
_YM = _YEARS * 6
_ROWS = _MD + _WH + _YM

_CHUNK = 256


def _build_pairs(tbl_v, a_v, b_v, dst_off, nb, count):
    def body(r, _):
        i = r // nb
        j = r - i * nb
        for k in range(_EMB // _L):
            va = a_v[pl.ds(i * _EMB + k * _L, _L)]
            vb = b_v[pl.ds(j * _EMB + k * _L, _L)]
            tbl_v[pl.ds((dst_off + r) * _EMB + k * _L, _L)] = va + vb
        return 0

    lax.fori_loop(0, count, body, 0)


def _sc_lookup(x_flat, yw, mw, dw, wdw, hw, nw):
    npos = x_flat.shape[0] // 7
    per_w = npos // _NW
    nchunk = per_w // _CHUNK
    mesh = plsc.VectorSubcoreMesh(core_axis_name="c", subcore_axis_name="s")

    @functools.partial(
        pl.kernel,
        mesh=mesh,
        compiler_params=pltpu.CompilerParams(needs_layout_passes=False),
        out_type=jax.ShapeDtypeStruct((npos * _EMB,), jnp.float32),
        scratch_types=[
            pltpu.VMEM((_ROWS * _EMB,), jnp.float32),
            pltpu.VMEM((_YEARS * _EMB,), jnp.float32),
            pltpu.VMEM((12 * _EMB,), jnp.float32),
            pltpu.VMEM((31 * _EMB,), jnp.float32),
            pltpu.VMEM((7 * _EMB,), jnp.float32),
            pltpu.VMEM((24 * _EMB,), jnp.float32),
            pltpu.VMEM((6 * _EMB,), jnp.float32),
            pltpu.VMEM((_CHUNK * 7,), jnp.int32),
            pltpu.VMEM((_CHUNK * _EMB,), jnp.float32),
        ],
    )
    def body(x_hbm, yw_hbm, mw_hbm, dw_hbm, wdw_hbm, hw_hbm, nw_hbm, out_hbm,
             tbl_v, yv, mv, dv, wv, hv, nv, x_v, o_v):
        wid = lax.axis_index("s") * _NC + lax.axis_index("c")
        base = wid * per_w

        pltpu.sync_copy(yw_hbm, yv)
        pltpu.sync_copy(mw_hbm, mv)
        pltpu.sync_copy(dw_hbm, dv)
        pltpu.sync_copy(wdw_hbm, wv)
        pltpu.sync_copy(hw_hbm, hv)
        pltpu.sync_copy(nw_hbm, nv)

        _build_pairs(tbl_v, mv, dv, 0, 31, _MD)
        _build_pairs(tbl_v, wv, hv, _MD, 24, _WH)
        _build_pairs(tbl_v, yv, nv, _MD + _WH, 6, _YM)

        lane = lax.iota(jnp.int32, _L)
        zero = jnp.int32(0)

        def fill_uniform(y0, m0, d0, w0, h0, n0):
            in_range = (y0 >= _YEAR0) & (y0 <= _YEAR0 + _YEARS - 1)
            yi = jnp.where(in_range, y0 - _YEAR0, y0)
            yi = jnp.minimum(jnp.maximum(yi, 0), _YEARS - 1)
            mi = jnp.minimum(jnp.maximum(m0 - 1, 0), 11)
            di = jnp.minimum(jnp.maximum(d0 - 1, 0), 30)
            wi = jnp.minimum(jnp.maximum(w0, 0), 6)
            hi = jnp.minimum(jnp.maximum(h0, 0), 23)
            ni = jnp.minimum(jnp.maximum(lax.div(n0, 10), 0), 5)
            md = (mi * 31 + di) * _EMB
            wh = (wi * 24 + hi + _MD) * _EMB
            ym = (yi * 6 + ni + _MD + _WH) * _EMB
            rows = [tbl_v[pl.ds(md + k * _L, _L)]
                    + tbl_v[pl.ds(wh + k * _L, _L)]
                    + tbl_v[pl.ds(ym + k * _L, _L)]
                    for k in range(_EMB // _L)]

            @plsc.parallel_loop(0, _CHUNK, 1, unroll=4)
            def fill_body(p):
                for k in range(_EMB // _L):
                    o_v[pl.ds(p * _EMB + k * _L, _L)] = rows[k]

        def fill_general():
            for g in range(_CHUNK // _L):
                p7 = (g * _L) * 7 + lane * 7
                year = plsc.load_gather(x_v, [p7])
                month = plsc.load_gather(x_v, [p7 + 1])
                day = plsc.load_gather(x_v, [p7 + 2])
                wday = plsc.load_gather(x_v, [p7 + 3])
                hour = plsc.load_gather(x_v, [p7 + 4])
                minute = plsc.load_gather(x_v, [p7 + 5])
                in_range = (year >= _YEAR0) & (year <= _YEAR0 + _YEARS - 1)
                yi = jnp.where(in_range, year - _YEAR0, year)
                yi = jnp.minimum(jnp.maximum(yi, 0), _YEARS - 1)
                mi = jnp.minimum(jnp.maximum(month - 1, 0), 11)
                di = jnp.minimum(jnp.maximum(day - 1, 0), 30)
                wi = jnp.minimum(jnp.maximum(wday, 0), 6)
                hi = jnp.minimum(jnp.maximum(hour, 0), 23)
                ni = jnp.minimum(jnp.maximum(lax.div(minute, 10), 0), 5)
                md = (mi * 31 + di) * _EMB
                wh = (wi * 24 + hi + _MD) * _EMB
                ym = (yi * 6 + ni + _MD + _WH) * _EMB
                ob = (g * _L + lane) * _EMB

                @plsc.parallel_loop(0, _EMB, 1, unroll=8)
                def col_body(c):
                    v = (plsc.load_gather(tbl_v, [md + c])
                         + plsc.load_gather(tbl_v, [wh + c])
                         + plsc.load_gather(tbl_v, [ym + c]))
                    plsc.store_scatter(o_v, [ob + c], v)

        def chunk_body(t, carry):
            y0p, m0p, d0p, w0p, h0p, n0p, valid = carry
            pos0 = base + t * _CHUNK
            pltpu.sync_copy(x_hbm.at[pl.ds(pos0 * 7, _CHUNK * 7)], x_v)
            rec0 = x_v[pl.ds(0, _L)]
            y0 = rec0[0]
            m0 = rec0[1]
            d0 = rec0[2]
            w0 = rec0[3]
            h0 = rec0[4]
            n0 = rec0[5]
            acc = lane < _L
            for g in range(_CHUNK // _L):
                p7 = (g * _L) * 7 + lane * 7
                eq = ((plsc.load_gather(x_v, [p7]) == y0)
                      & (plsc.load_gather(x_v, [p7 + 1]) == m0)
                      & (plsc.load_gather(x_v, [p7 + 2]) == d0)
                      & (plsc.load_gather(x_v, [p7 + 3]) == w0)
                      & (plsc.load_gather(x_v, [p7 + 4]) == h0)
                      & (plsc.load_gather(x_v, [p7 + 5]) == n0))
                acc = acc & eq
            uniform = jnp.all(acc)
            same = (uniform & (valid == 1)
                    & (y0 == y0p) & (m0 == m0p) & (d0 == d0p)
                    & (w0 == w0p) & (h0 == h0p) & (n0 == n0p))

            def stale():
                lax.cond(uniform,
                         lambda: fill_uniform(y0, m0, d0, w0, h0, n0),
                         fill_general)

            lax.cond(same, lambda: None, stale)
            pltpu.sync_copy(o_v, out_hbm.at[pl.ds(pos0 * _EMB, _CHUNK * _EMB)])
            return (y0, m0, d0, w0, h0, n0,
                    jnp.where(uniform, jnp.int32(1), zero))

        lax.fori_loop(0, nchunk, chunk_body,
                      (zero, zero, zero, zero, zero, zero, zero))

    return body(x_flat, yw, mw, dw, wdw, hw, nw)


def kernel(x, year_w, month_w, day_w, weekday_w, hour_w, min_w):
    b, l, _ = x.shape
    out_flat = _sc_lookup(
        x.reshape(-1),
        year_w.reshape(-1), month_w.reshape(-1), day_w.reshape(-1),
        weekday_w.reshape(-1), hour_w.reshape(-1), min_w.reshape(-1),
    )
    return out_flat.reshape(b, l, _EMB)


# speedup vs baseline: 30.8573x; 1.1437x over previous
import functools

import jax
import jax.numpy as jnp
from jax import lax
from jax.experimental import pallas as pl
from jax.experimental.pallas import tpu as pltpu
from jax.experimental.pallas import tpu_sc as plsc

_NC = 2
_NS = 16
_L = 16
_NW = _NC * _NS

_YEARS = 14
_YEAR0 = 2010
_EMB = 128

_MD = 12 * 31
_WH = 7 * 24
_YM = _YEARS * 6
_ROWS = _MD + _WH + _YM

_CHUNK = 256


def _build_pairs(tbl_v, a_v, b_v, dst_off, nb, count):
    @plsc.parallel_loop(0, count, 1, unroll=2)
    def body(r):
        i = r // nb
        j = r - i * nb
        for k in range(_EMB // _L):
            va = a_v[pl.ds(i * _EMB + k * _L, _L)]
            vb = b_v[pl.ds(j * _EMB + k * _L, _L)]
            tbl_v[pl.ds((dst_off + r) * _EMB + k * _L, _L)] = va + vb


def _sc_lookup(x_flat, yw, mw, dw, wdw, hw, nw):
    npos = x_flat.shape[0] // 7
    per_w = npos // _NW
    nchunk = per_w // _CHUNK
    mesh = plsc.VectorSubcoreMesh(core_axis_name="c", subcore_axis_name="s")

    @functools.partial(
        pl.kernel,
        mesh=mesh,
        compiler_params=pltpu.CompilerParams(needs_layout_passes=False),
        out_type=jax.ShapeDtypeStruct((npos * _EMB,), jnp.float32),
        scratch_types=[
            pltpu.VMEM((_ROWS * _EMB,), jnp.float32),
            pltpu.VMEM((_YEARS * _EMB,), jnp.float32),
            pltpu.VMEM((12 * _EMB,), jnp.float32),
            pltpu.VMEM((31 * _EMB,), jnp.float32),
            pltpu.VMEM((7 * _EMB,), jnp.float32),
            pltpu.VMEM((24 * _EMB,), jnp.float32),
            pltpu.VMEM((6 * _EMB,), jnp.float32),
            pltpu.VMEM((_CHUNK * 7,), jnp.int32),
            pltpu.VMEM((_CHUNK * 7,), jnp.int32),
            pltpu.VMEM((_CHUNK * _EMB,), jnp.float32),
            pltpu.SemaphoreType.DMA,
            pltpu.SemaphoreType.DMA,
            pltpu.SemaphoreType.DMA,
        ],
    )
    def body(x_hbm, yw_hbm, mw_hbm, dw_hbm, wdw_hbm, hw_hbm, nw_hbm, out_hbm,
             tbl_v, yv, mv, dv, wv, hv, nv, xa_v, xb_v, o_v,
             sem_xa, sem_xb, sem_o):
        wid = lax.axis_index("s") * _NC + lax.axis_index("c")
        base = wid * per_w

        pltpu.sync_copy(yw_hbm, yv)
        pltpu.sync_copy(mw_hbm, mv)
        pltpu.sync_copy(dw_hbm, dv)
        pltpu.sync_copy(wdw_hbm, wv)
        pltpu.sync_copy(hw_hbm, hv)
        pltpu.sync_copy(nw_hbm, nv)

        _build_pairs(tbl_v, mv, dv, 0, 31, _MD)
        _build_pairs(tbl_v, wv, hv, _MD, 24, _WH)
        _build_pairs(tbl_v, yv, nv, _MD + _WH, 6, _YM)

        lane = lax.iota(jnp.int32, _L)
        zero = jnp.int32(0)

        def fill_uniform(y0, m0, d0, w0, h0, n0):
            in_range = (y0 >= _YEAR0) & (y0 <= _YEAR0 + _YEARS - 1)
            yi = jnp.where(in_range, y0 - _YEAR0, y0)
            yi = jnp.minimum(jnp.maximum(yi, 0), _YEARS - 1)
            mi = jnp.minimum(jnp.maximum(m0 - 1, 0), 11)
            di = jnp.minimum(jnp.maximum(d0 - 1, 0), 30)
            wi = jnp.minimum(jnp.maximum(w0, 0), 6)
            hi = jnp.minimum(jnp.maximum(h0, 0), 23)
            ni = jnp.minimum(jnp.maximum(lax.div(n0, 10), 0), 5)
            md = (mi * 31 + di) * _EMB
            wh = (wi * 24 + hi + _MD) * _EMB
            ym = (yi * 6 + ni + _MD + _WH) * _EMB
            rows = [tbl_v[pl.ds(md + k * _L, _L)]
                    + tbl_v[pl.ds(wh + k * _L, _L)]
                    + tbl_v[pl.ds(ym + k * _L, _L)]
                    for k in range(_EMB // _L)]

            @plsc.parallel_loop(0, _CHUNK, 1, unroll=4)
            def fill_body(p):
                for k in range(_EMB // _L):
                    o_v[pl.ds(p * _EMB + k * _L, _L)] = rows[k]

        def fill_general(x_v):
            for g in range(_CHUNK // _L):
                p7 = (g * _L) * 7 + lane * 7
                year = plsc.load_gather(x_v, [p7])
                month = plsc.load_gather(x_v, [p7 + 1])
                day = plsc.load_gather(x_v, [p7 + 2])
                wday = plsc.load_gather(x_v, [p7 + 3])
                hour = plsc.load_gather(x_v, [p7 + 4])
                minute = plsc.load_gather(x_v, [p7 + 5])
                in_range = (year >= _YEAR0) & (year <= _YEAR0 + _YEARS - 1)
                yi = jnp.where(in_range, year - _YEAR0, year)
                yi = jnp.minimum(jnp.maximum(yi, 0), _YEARS - 1)
                mi = jnp.minimum(jnp.maximum(month - 1, 0), 11)
                di = jnp.minimum(jnp.maximum(day - 1, 0), 30)
                wi = jnp.minimum(jnp.maximum(wday, 0), 6)
                hi = jnp.minimum(jnp.maximum(hour, 0), 23)
                ni = jnp.minimum(jnp.maximum(lax.div(minute, 10), 0), 5)
                md = (mi * 31 + di) * _EMB
                wh = (wi * 24 + hi + _MD) * _EMB
                ym = (yi * 6 + ni + _MD + _WH) * _EMB
                ob = (g * _L + lane) * _EMB

                @plsc.parallel_loop(0, _EMB, 1, unroll=8)
                def col_body(c):
                    v = (plsc.load_gather(tbl_v, [md + c])
                         + plsc.load_gather(tbl_v, [wh + c])
                         + plsc.load_gather(tbl_v, [ym + c]))
                    plsc.store_scatter(o_v, [ob + c], v)

        def x_copy(t, x_v, sem):
            return pltpu.make_async_copy(
                x_hbm.at[pl.ds((base + t * _CHUNK) * 7, _CHUNK * 7)], x_v, sem)

        def o_copy(t):
            return pltpu.make_async_copy(
                o_v, out_hbm.at[pl.ds((base + t * _CHUNK) * _EMB,
                                      _CHUNK * _EMB)], sem_o)

        def process(t, x_v, x_nxt, sem_nxt, carry):
            y0p, m0p, d0p, w0p, h0p, n0p, valid = carry

            @pl.when(t + 1 < nchunk)
            def _():
                x_copy(t + 1, x_nxt, sem_nxt).start()

            rec0 = x_v[pl.ds(0, _L)]
            y0 = rec0[0]
            m0 = rec0[1]
            d0 = rec0[2]
            w0 = rec0[3]
            h0 = rec0[4]
            n0 = rec0[5]
            acc = lane < _L
            for g in range(_CHUNK // _L):
                p7 = (g * _L) * 7 + lane * 7
                eq = ((plsc.load_gather(x_v, [p7]) == y0)
                      & (plsc.load_gather(x_v, [p7 + 1]) == m0)
                      & (plsc.load_gather(x_v, [p7 + 2]) == d0)
                      & (plsc.load_gather(x_v, [p7 + 3]) == w0)
                      & (plsc.load_gather(x_v, [p7 + 4]) == h0)
                      & (plsc.load_gather(x_v, [p7 + 5]) == n0))
                acc = acc & eq
            uniform = jnp.all(acc)
            same = (uniform & (valid == 1)
                    & (y0 == y0p) & (m0 == m0p) & (d0 == d0p)
                    & (w0 == w0p) & (h0 == h0p) & (n0 == n0p))

            @pl.when(t > 0)
            def _():
                o_copy(t - 1).wait()

            def stale():
                lax.cond(uniform,
                         lambda: fill_uniform(y0, m0, d0, w0, h0, n0),
                         lambda: fill_general(x_v))

            lax.cond(same, lambda: None, stale)
            o_copy(t).start()
            return (y0, m0, d0, w0, h0, n0,
                    jnp.where(uniform, jnp.int32(1), zero))

        x_copy(0, xa_v, sem_xa).start()

        def pair_body(i, carry):
            t = i * 2
            x_copy(t, xa_v, sem_xa).wait()
            carry = process(t, xa_v, xb_v, sem_xb, carry)
            x_copy(t + 1, xb_v, sem_xb).wait()
            carry = process(t + 1, xb_v, xa_v, sem_xa, carry)
            return carry

        lax.fori_loop(0, nchunk // 2, pair_body,
                      (zero, zero, zero, zero, zero, zero, zero))
        o_copy(nchunk - 1).wait()

    return body(x_flat, yw, mw, dw, wdw, hw, nw)


def kernel(x, year_w, month_w, day_w, weekday_w, hour_w, min_w):
    b, l, _ = x.shape
    out_flat = _sc_lookup(
        x.reshape(-1),
        year_w.reshape(-1), month_w.reshape(-1), day_w.reshape(-1),
        weekday_w.reshape(-1), hour_w.reshape(-1), min_w.reshape(-1),
    )
    return out_flat.reshape(b, l, _EMB)
